# initial kernel scaffold (unmeasured)
import functools

import jax
import jax.numpy as jnp
from jax import lax
from jax.experimental import pallas as pl
from jax.experimental.pallas import tpu as pltpu

N_DEV = 4
B = 2
SQL = 128
HQ = 4
DH = 64
HD = HQ * DH
SKV = N_DEV * SQL
DM = 512
BLK = 64
BF16 = jnp.bfloat16
MESH = pl.DeviceIdType.MESH


def kernel(x, Wq, K_ext, V_ext, Wo):
    def body(x_ref, wq_ref, k_ref, v_ref, wo_ref, out_ref,
             kvbuf, send_sems, recv_sems):
        me = lax.axis_index("i")

        kvbuf[pl.ds(me, 1), 0] = k_ref[:].reshape(B, SQL, HD).astype(BF16)[None]
        kvbuf[pl.ds(me, 1), 1] = v_ref[:].reshape(B, SQL, HD).astype(BF16)[None]

        bar = pltpu.get_barrier_semaphore()
        for k in range(1, N_DEV):
            pl.semaphore_signal(bar, inc=1, device_id=((me + k) % N_DEV,),
                                device_id_type=MESH)
        pl.semaphore_wait(bar, N_DEV - 1)

        def pair_rdma(src, dst):
            return pltpu.make_async_remote_copy(
                src_ref=kvbuf.at[src],
                dst_ref=kvbuf.at[src],
                send_sem=send_sems.at[dst],
                recv_sem=recv_sems.at[src],
                device_id=(dst,),
                device_id_type=MESH,
            )

        for src in range(N_DEV - 1):
            @pl.when(me == src)
            def _(src=src):
                for dst in range(src + 1, N_DEV):
                    pair_rdma(src, dst).start()

        wq = wq_ref[:].astype(BF16)
        q = [
            lax.dot_general(x_ref[b].astype(BF16), wq, (((1,), (0,)), ((), ())),
                            preferred_element_type=jnp.float32)
            for b in range(B)
        ]

        for src in range(N_DEV - 1):
            @pl.when(me > src)
            def _(src=src):
                pair_rdma(src, src + 1).wait_recv()

        kb = lax.broadcasted_iota(jnp.int32, (SQL, SKV), 1) // BLK
        qb = 2 * me + lax.broadcasted_iota(jnp.int32, (SQL, SKV), 0) // BLK
        mask = kb <= qb

        wo = wo_ref[:].astype(BF16)
        for b in range(B):
            ctx_heads = []
            for h in range(HQ):
                qh = q[b][:, h * DH:(h + 1) * DH].astype(BF16)
                kf = kvbuf[:, 0, b, :, h * DH:(h + 1) * DH].reshape(SKV, DH)
                vf = kvbuf[:, 1, b, :, h * DH:(h + 1) * DH].reshape(SKV, DH)
                s = lax.dot_general(qh, kf, (((1,), (1,)), ((), ())),
                                    preferred_element_type=jnp.float32) * 0.125
                s = jnp.where(mask, s, -1e9)
                m = jnp.max(s, axis=1, keepdims=True)
                w = jnp.exp(s - m)
                w = w / jnp.sum(w, axis=1, keepdims=True)
                ctx_heads.append(
                    lax.dot_general(w.astype(BF16), vf, (((1,), (0,)), ((), ())),
                                    preferred_element_type=jnp.float32))
            ctx = jnp.concatenate(ctx_heads, axis=1).astype(BF16)
            out_ref[b] = lax.dot_general(ctx, wo, (((1,), (0,)), ((), ())),
                                         preferred_element_type=jnp.float32)

        for src in range(N_DEV - 1):
            @pl.when(me == src)
            def _(src=src):
                for dst in range(src + 1, N_DEV):
                    pair_rdma(src, dst).wait_send()

        @functools.partial(pl.run_scoped, sem2=pltpu.SemaphoreType.REGULAR)
        def _(sem2):
            for k in range(1, N_DEV):
                pl.semaphore_signal(sem2, inc=1, device_id=((me + k) % N_DEV,),
                                    device_id_type=MESH)
            pl.semaphore_wait(sem2, N_DEV - 1)

    return pl.pallas_call(
        body,
        out_shape=jax.ShapeDtypeStruct((B, SQL, DM), jnp.float32),
        in_specs=[pl.BlockSpec(memory_space=pltpu.VMEM)] * 5,
        out_specs=pl.BlockSpec(memory_space=pltpu.VMEM),
        scratch_shapes=[
            pltpu.VMEM((N_DEV, 2, B, SQL, HD), BF16),
            pltpu.SemaphoreType.DMA((N_DEV,)),
            pltpu.SemaphoreType.DMA((N_DEV,)),
        ],
        compiler_params=pltpu.CompilerParams(collective_id=0),
    )(x, Wq, K_ext, V_ext, Wo)


# baseline (device time: 19216 ns/iter reference)
import functools

import jax
import jax.numpy as jnp
from jax import lax
from jax.experimental import pallas as pl
from jax.experimental.pallas import tpu as pltpu

N_DEV = 4
B = 2
SQL = 128
HQ = 4
DH = 64
HD = HQ * DH
SKV = N_DEV * SQL
DM = 512
BLK = 64
BF16 = jnp.bfloat16
MESH = pl.DeviceIdType.MESH


def kernel(x, Wq, K_ext, V_ext, Wo):
    def body(x_ref, wq_ref, k_ref, v_ref, wo_ref, out_ref,
             kvbuf, send_sems, recv_sems):
        me = lax.axis_index("i")

        kvbuf[...] = jnp.zeros((N_DEV, 2, B, SQL, HD), BF16)

        kvbuf[pl.ds(me, 1), 0] = k_ref[:].reshape(B, SQL, HD).astype(BF16)[None]
        kvbuf[pl.ds(me, 1), 1] = v_ref[:].reshape(B, SQL, HD).astype(BF16)[None]

        bar = pltpu.get_barrier_semaphore()
        for k in range(1, N_DEV):
            pl.semaphore_signal(bar, inc=1, device_id=((me + k) % N_DEV,),
                                device_id_type=MESH)
        pl.semaphore_wait(bar, N_DEV - 1)

        def pair_rdma(src, dst):
            return pltpu.make_async_remote_copy(
                src_ref=kvbuf.at[src],
                dst_ref=kvbuf.at[src],
                send_sem=send_sems.at[dst],
                recv_sem=recv_sems.at[src],
                device_id=(dst,),
                device_id_type=MESH,
            )

        for src in range(N_DEV - 1):
            @pl.when(me == src)
            def _(src=src):
                for dst in range(src + 1, N_DEV):
                    pair_rdma(src, dst).start()

        wq = wq_ref[:].astype(BF16)
        q = [
            lax.dot_general(x_ref[b].astype(BF16), wq, (((1,), (0,)), ((), ())),
                            preferred_element_type=jnp.float32)
            for b in range(B)
        ]

        for src in range(N_DEV - 1):
            @pl.when(me > src)
            def _(src=src):
                pair_rdma(src, src + 1).wait_recv()

        kb = lax.broadcasted_iota(jnp.int32, (SQL, SKV), 1) // BLK
        qb = 2 * me + lax.broadcasted_iota(jnp.int32, (SQL, SKV), 0) // BLK
        mask = kb <= qb

        wo = wo_ref[:].astype(BF16)
        for b in range(B):
            ctx_heads = []
            for h in range(HQ):
                qh = q[b][:, h * DH:(h + 1) * DH].astype(BF16)
                kf = kvbuf[:, 0, b, :, h * DH:(h + 1) * DH].reshape(SKV, DH)
                vf = kvbuf[:, 1, b, :, h * DH:(h + 1) * DH].reshape(SKV, DH)
                s = lax.dot_general(qh, kf, (((1,), (1,)), ((), ())),
                                    preferred_element_type=jnp.float32) * 0.125
                s = jnp.where(mask, s, -1e9)
                m = jnp.max(s, axis=1, keepdims=True)
                w = jnp.exp(s - m)
                w = w / jnp.sum(w, axis=1, keepdims=True)
                ctx_heads.append(
                    lax.dot_general(w.astype(BF16), vf, (((1,), (0,)), ((), ())),
                                    preferred_element_type=jnp.float32))
            ctx = jnp.concatenate(ctx_heads, axis=1).astype(BF16)
            out_ref[b] = lax.dot_general(ctx, wo, (((1,), (0,)), ((), ())),
                                         preferred_element_type=jnp.float32)

        for src in range(N_DEV - 1):
            @pl.when(me == src)
            def _(src=src):
                for dst in range(src + 1, N_DEV):
                    pair_rdma(src, dst).wait_send()

        @functools.partial(pl.run_scoped, sem2=pltpu.SemaphoreType.REGULAR)
        def _(sem2):
            for k in range(1, N_DEV):
                pl.semaphore_signal(sem2, inc=1, device_id=((me + k) % N_DEV,),
                                    device_id_type=MESH)
            pl.semaphore_wait(sem2, N_DEV - 1)

    return pl.pallas_call(
        body,
        out_shape=jax.ShapeDtypeStruct((B, SQL, DM), jnp.float32),
        in_specs=[pl.BlockSpec(memory_space=pltpu.VMEM)] * 5,
        out_specs=pl.BlockSpec(memory_space=pltpu.VMEM),
        scratch_shapes=[
            pltpu.VMEM((N_DEV, 2, B, SQL, HD), BF16),
            pltpu.SemaphoreType.DMA((N_DEV,)),
            pltpu.SemaphoreType.DMA((N_DEV,)),
        ],
        compiler_params=pltpu.CompilerParams(collective_id=0),
    )(x, Wq, K_ext, V_ext, Wo)


# device time: 16188 ns/iter; 1.1871x vs baseline; 1.1871x over previous
import jax
import jax.numpy as jnp
from jax import lax
from jax.experimental import pallas as pl
from jax.experimental.pallas import tpu as pltpu

N_DEV = 4
B = 2
SQL = 128
HQ = 4
DH = 64
HD = HQ * DH
SKV = N_DEV * SQL
DM = 512
BLK = 64
BF16 = jnp.bfloat16
MESH = pl.DeviceIdType.MESH

_DST_ORDER = {0: (1, 3, 2), 1: (2, 3), 2: (3,)}


def kernel(x, Wq, K_ext, V_ext, Wo):
    def body(x_ref, wq_ref, k_ref, v_ref, wo_ref, out_ref,
             kvbuf, send_sems, recv_sems, cred_sems):
        me = lax.axis_index("i")

        bar = pltpu.get_barrier_semaphore()
        for k in range(1, N_DEV):
            pl.semaphore_signal(bar, inc=1, device_id=((me + k) % N_DEV,),
                                device_id_type=MESH)
        pl.semaphore_wait(bar, N_DEV - 1)

        for d in range(1, N_DEV):
            @pl.when(me == d)
            def _(d=d):
                for o in range(d):
                    pl.semaphore_signal(cred_sems.at[d], inc=1,
                                        device_id=(o,), device_id_type=MESH)

        kvbuf[pl.ds(me, 1), 0] = k_ref[:].reshape(B, SQL, HD).astype(BF16)[None]
        kvbuf[pl.ds(me, 1), 1] = v_ref[:].reshape(B, SQL, HD).astype(BF16)[None]

        def pair_rdma(src, dst):
            return pltpu.make_async_remote_copy(
                src_ref=kvbuf.at[src],
                dst_ref=kvbuf.at[src],
                send_sem=send_sems.at[dst],
                recv_sem=recv_sems.at[src],
                device_id=(dst,),
                device_id_type=MESH,
            )

        for src, dsts in _DST_ORDER.items():
            @pl.when(me == src)
            def _(src=src, dsts=dsts):
                for dst in dsts:
                    pl.semaphore_wait(cred_sems.at[dst], 1)
                    pair_rdma(src, dst).start()

        wq = wq_ref[:].astype(BF16)
        q = [
            lax.dot_general(x_ref[b].astype(BF16), wq, (((1,), (0,)), ((), ())),
                            preferred_element_type=jnp.float32)
            for b in range(B)
        ]

        s = [[[None] * N_DEV for _ in range(HQ)] for _ in range(B)]
        for src in range(N_DEV):
            if src < N_DEV - 1:
                @pl.when(me > src)
                def _(src=src):
                    pair_rdma(src, src + 1).wait_recv()
            for b in range(B):
                for h in range(HQ):
                    qh = q[b][:, h * DH:(h + 1) * DH].astype(BF16)
                    kf = kvbuf[src, 0, b, :, h * DH:(h + 1) * DH]
                    s[b][h][src] = lax.dot_general(
                        qh, kf, (((1,), (1,)), ((), ())),
                        preferred_element_type=jnp.float32)

        kb = lax.broadcasted_iota(jnp.int32, (SQL, SKV), 1) // BLK
        qb = 2 * me + lax.broadcasted_iota(jnp.int32, (SQL, SKV), 0) // BLK
        mask = kb <= qb
        colvis = (lax.broadcasted_iota(jnp.int32, (SKV, 1), 0) // BLK) <= 2 * me + 1

        wo = wo_ref[:].astype(BF16)
        for b in range(B):
            ctx_heads = []
            for h in range(HQ):
                sf = jnp.concatenate(s[b][h], axis=1) * 0.125
                sf = jnp.where(mask, sf, -1e9)
                m = jnp.max(sf, axis=1, keepdims=True)
                w = jnp.exp(sf - m)
                w = w / jnp.sum(w, axis=1, keepdims=True)
                vf = kvbuf[:, 1, b, :, h * DH:(h + 1) * DH].reshape(SKV, DH)
                vf = jnp.where(colvis, vf, jnp.zeros((), BF16))
                ctx_heads.append(
                    lax.dot_general(w.astype(BF16), vf, (((1,), (0,)), ((), ())),
                                    preferred_element_type=jnp.float32))
            ctx = jnp.concatenate(ctx_heads, axis=1).astype(BF16)
            out_ref[b] = lax.dot_general(ctx, wo, (((1,), (0,)), ((), ())),
                                         preferred_element_type=jnp.float32)

        for src, dsts in _DST_ORDER.items():
            @pl.when(me == src)
            def _(src=src, dsts=dsts):
                for dst in dsts:
                    pair_rdma(src, dst).wait_send()

    return pl.pallas_call(
        body,
        out_shape=jax.ShapeDtypeStruct((B, SQL, DM), jnp.float32),
        in_specs=[pl.BlockSpec(memory_space=pltpu.VMEM)] * 5,
        out_specs=pl.BlockSpec(memory_space=pltpu.VMEM),
        scratch_shapes=[
            pltpu.VMEM((N_DEV, 2, B, SQL, HD), BF16),
            pltpu.SemaphoreType.DMA((N_DEV,)),
            pltpu.SemaphoreType.DMA((N_DEV,)),
            pltpu.SemaphoreType.REGULAR((N_DEV,)),
        ],
        compiler_params=pltpu.CompilerParams(collective_id=0),
    )(x, Wq, K_ext, V_ext, Wo)


# device time: 15833 ns/iter; 1.2137x vs baseline; 1.0224x over previous
import jax
import jax.numpy as jnp
from jax import lax
from jax.experimental import pallas as pl
from jax.experimental.pallas import tpu as pltpu

N_DEV = 4
B = 2
SQL = 128
HQ = 4
DH = 64
HD = HQ * DH
SKV = N_DEV * SQL
DM = 512
BLK = 64
BF16 = jnp.bfloat16
MESH = pl.DeviceIdType.MESH

_DST_ORDER = {0: (1, 3, 2), 1: (2, 3), 2: (3,)}


def kernel(x, Wq, K_ext, V_ext, Wo):
    def body(x_hbm, wq_hbm, k_hbm, v_hbm, wo_hbm, out_ref,
             kvbuf, send_sems, recv_sems, cred_sems,
             xv, wqv, kv, vv, wov, copy_sems):
        me = lax.axis_index("i")

        bar = pltpu.get_barrier_semaphore()
        for k in range(1, N_DEV):
            pl.semaphore_signal(bar, inc=1, device_id=((me + k) % N_DEV,),
                                device_id_type=MESH)
        pl.semaphore_wait(bar, N_DEV - 1)

        copies = [
            pltpu.make_async_copy(s_, d_, copy_sems.at[i])
            for i, (s_, d_) in enumerate(
                [(k_hbm, kv), (v_hbm, vv), (x_hbm, xv), (wq_hbm, wqv),
                 (wo_hbm, wov)])
        ]
        for c in copies:
            c.start()

        for d in range(1, N_DEV):
            @pl.when(me == d)
            def _(d=d):
                for o in range(d):
                    pl.semaphore_signal(cred_sems.at[d], inc=1,
                                        device_id=(o,), device_id_type=MESH)

        copies[0].wait()
        copies[1].wait()
        kvbuf[pl.ds(me, 1), 0] = kv[:].reshape(B, SQL, HD).astype(BF16)[None]
        kvbuf[pl.ds(me, 1), 1] = vv[:].reshape(B, SQL, HD).astype(BF16)[None]

        def pair_rdma(src, dst):
            return pltpu.make_async_remote_copy(
                src_ref=kvbuf.at[src],
                dst_ref=kvbuf.at[src],
                send_sem=send_sems.at[dst],
                recv_sem=recv_sems.at[src],
                device_id=(dst,),
                device_id_type=MESH,
            )

        for src, dsts in _DST_ORDER.items():
            @pl.when(me == src)
            def _(src=src, dsts=dsts):
                for dst in dsts:
                    pl.semaphore_wait(cred_sems.at[dst], 1)
                    pair_rdma(src, dst).start()

        copies[2].wait()
        copies[3].wait()
        wq = wqv[:].astype(BF16)
        q = [
            lax.dot_general(xv[b].astype(BF16), wq, (((1,), (0,)), ((), ())),
                            preferred_element_type=jnp.float32)
            for b in range(B)
        ]

        qrow = lax.broadcasted_iota(jnp.int32, (SQL, SQL), 0) // BLK
        krow = lax.broadcasted_iota(jnp.int32, (SQL, SQL), 1) // BLK
        acc = [[jnp.zeros((SQL, DH), jnp.float32) for _ in range(HQ)]
               for _ in range(B)]
        lse = [[jnp.zeros((SQL, 1), jnp.float32) for _ in range(HQ)]
               for _ in range(B)]
        for src in range(N_DEV):
            if src < N_DEV - 1:
                @pl.when(me > src)
                def _(src=src):
                    pair_rdma(src, src + 1).wait_recv()
            smask = (2 * src + krow) <= (2 * me + qrow)
            vis = src <= me
            for b in range(B):
                for h in range(HQ):
                    qh = q[b][:, h * DH:(h + 1) * DH].astype(BF16)
                    kf = kvbuf[src, 0, b, :, h * DH:(h + 1) * DH]
                    vf = kvbuf[src, 1, b, :, h * DH:(h + 1) * DH]
                    vf = jnp.where(vis, vf, jnp.zeros((), BF16))
                    s = lax.dot_general(qh, kf, (((1,), (1,)), ((), ())),
                                        preferred_element_type=jnp.float32)
                    p = jnp.exp(jnp.where(smask, s * 0.125, -1e9))
                    lse[b][h] = lse[b][h] + jnp.sum(p, axis=1, keepdims=True)
                    acc[b][h] = acc[b][h] + lax.dot_general(
                        p.astype(BF16), vf, (((1,), (0,)), ((), ())),
                        preferred_element_type=jnp.float32)

        copies[4].wait()
        wo = wov[:].astype(BF16)
        for b in range(B):
            ctx = jnp.concatenate(
                [acc[b][h] / lse[b][h] for h in range(HQ)], axis=1)
            out_ref[b] = lax.dot_general(ctx.astype(BF16), wo,
                                         (((1,), (0,)), ((), ())),
                                         preferred_element_type=jnp.float32)

        for src, dsts in _DST_ORDER.items():
            @pl.when(me == src)
            def _(src=src, dsts=dsts):
                for dst in dsts:
                    pair_rdma(src, dst).wait_send()

    return pl.pallas_call(
        body,
        out_shape=jax.ShapeDtypeStruct((B, SQL, DM), jnp.float32),
        in_specs=[pl.BlockSpec(memory_space=pl.ANY)] * 5,
        out_specs=pl.BlockSpec(memory_space=pltpu.VMEM),
        scratch_shapes=[
            pltpu.VMEM((N_DEV, 2, B, SQL, HD), BF16),
            pltpu.SemaphoreType.DMA((N_DEV,)),
            pltpu.SemaphoreType.DMA((N_DEV,)),
            pltpu.SemaphoreType.REGULAR((N_DEV,)),
            pltpu.VMEM((B, SQL, DM), jnp.float32),
            pltpu.VMEM((DM, HD), jnp.float32),
            pltpu.VMEM((B, SQL, HQ, DH), jnp.float32),
            pltpu.VMEM((B, SQL, HQ, DH), jnp.float32),
            pltpu.VMEM((HD, DM), jnp.float32),
            pltpu.SemaphoreType.DMA((5,)),
        ],
        compiler_params=pltpu.CompilerParams(collective_id=0),
    )(x, Wq, K_ext, V_ext, Wo)
